# baseline (speedup 1.0000x reference)
import jax
import jax.numpy as jnp
from jax import lax
from jax.experimental import pallas as pl
from jax.experimental.pallas import tpu as pltpu
from jax.experimental.pallas import tpu_sc as plsc

N = 10000
E = 160000
D = 256
NPAD = 10240
NC = 2
NS = 16
NW = NC * NS
TPW = NPAD // NW
ACCR = TPW + 8
CH = 1600
NCHE = E // CH
NV = CH // 16
CL = 6480
FB = 2048
SB = 64
RV = 8192
GB = 32
DEPTH = 4
LCAP = E + RV

BR = 400
GRID = N // BR


def _mesh():
    return plsc.VectorSubcoreMesh(core_axis_name="c", subcore_axis_name="s")


_SC_PARAMS = pltpu.CompilerParams(needs_layout_passes=False)


def _route_body(src_hbm, dst_hbm, lists_hbm, deg_hbm,
                sb0, db0, sb1, db1, pklist, hist, semA, semB):
    c = lax.axis_index("c")
    s = lax.axis_index("s")
    w = s * NC + c
    base = w * TPW
    lbase = w * LCAP

    iota16 = lax.iota(jnp.int32, 16)
    trash16 = TPW + (iota16 & 7)
    zeros16 = jnp.zeros((16,), jnp.float32)
    ones16 = jnp.ones((16,), jnp.float32)

    for j in range(21):
        hist[pl.ds(j * 16, 16)] = zeros16

    def stage_start(ch, sbuf, dbuf, sem):
        eoff = pl.multiple_of(ch * CH, 8)
        pltpu.async_copy(src_hbm.at[pl.ds(eoff, CH)], sbuf, sem)
        pltpu.async_copy(dst_hbm.at[pl.ds(eoff, CH)], dbuf, sem)

    def stage_wait(sbuf, dbuf, sem):
        pltpu.make_async_copy(src_hbm.at[pl.ds(0, CH)], sbuf, sem).wait()
        pltpu.make_async_copy(dst_hbm.at[pl.ds(0, CH)], dbuf, sem).wait()

    def scan_chunk(sbuf, dbuf, cnt0):
        def scan(i, cnt):
            d = dbuf[pl.ds(i * 16, 16)]
            sv = sbuf[pl.ds(i * 16, 16)]
            loc = d - base
            mask = (loc >= 0) & (loc < TPW)
            packed = jnp.bitwise_or(jnp.left_shift(sv, 9), loc & 511)
            plsc.store_compressed(pklist.at[pl.ds(cnt, 16)], packed, mask=mask)
            locm = jnp.where(mask, loc, TPW)
            plsc.addupdate_scatter(hist, [locm], ones16, mask=mask)
            pc = plsc.all_reduce_population_count(mask)
            return cnt + pc[0]

        return pl.loop(0, NV, init_carry=cnt0, unroll=4)(scan)

    def maybe_flush(m, tot):
        cond = m > CL - 2 * CH
        nfb = jnp.where(cond, jnp.right_shift(m, 11), 0)

        @pl.when(cond)
        def _():
            @pl.loop(0, nfb)
            def _f(k):
                off = pl.multiple_of(k * FB, 8)
                pltpu.sync_copy(pklist.at[pl.ds(off, FB)],
                                lists_hbm.at[pl.ds(pl.multiple_of(lbase + tot + off, 8), FB)])

            rem0 = nfb * FB
            nmv = jnp.right_shift((m - rem0) + 15, 4)

            @pl.loop(0, nmv)
            def _mv(j):
                pklist[pl.ds(pl.multiple_of(j * 16, 8), 16)] = (
                    pklist[pl.ds(pl.multiple_of(rem0 + j * 16, 8), 16)])

        return jnp.where(cond, m - nfb * FB, m), tot + nfb * FB

    stage_start(0, sb0, db0, semA)

    @pl.loop(0, NCHE // 2, init_carry=(jnp.int32(0), jnp.int32(0)))
    def _pair(p, carry):
        m, tot = carry
        ch0 = p * 2
        stage_wait(sb0, db0, semA)
        stage_start(ch0 + 1, sb1, db1, semB)
        m = scan_chunk(sb0, db0, m)
        stage_wait(sb1, db1, semB)

        @pl.when(ch0 + 2 < NCHE)
        def _():
            stage_start(ch0 + 2, sb0, db0, semA)

        m = scan_chunk(sb1, db1, m)
        return maybe_flush(m, tot)

    m, tot = _pair

    for kpad in range(SB // 16):
        pklist[pl.ds(m + kpad * 16, 16)] = trash16
    n64 = jnp.right_shift(m + (SB - 1), 6)

    @pl.loop(0, n64)
    def _ff(k):
        off = pl.multiple_of(k * SB, 8)
        pltpu.sync_copy(pklist.at[pl.ds(off, SB)],
                        lists_hbm.at[pl.ds(pl.multiple_of(lbase + tot + off, 8), SB)])

    pltpu.sync_copy(hist.at[pl.ds(0, TPW)],
                    deg_hbm.at[pl.ds(pl.multiple_of(base, 8), TPW)])


def _route_call(src, dst):
    return pl.kernel(
        _route_body,
        out_type=(jax.ShapeDtypeStruct((NW * LCAP,), jnp.int32),
                  jax.ShapeDtypeStruct((NPAD,), jnp.float32)),
        mesh=_mesh(),
        compiler_params=_SC_PARAMS,
        scratch_types=[
            pltpu.VMEM((CH,), jnp.int32),
            pltpu.VMEM((CH,), jnp.int32),
            pltpu.VMEM((CH,), jnp.int32),
            pltpu.VMEM((CH,), jnp.int32),
            pltpu.VMEM((CL,), jnp.int32),
            pltpu.VMEM((336,), jnp.float32),
            pltpu.SemaphoreType.DMA,
            pltpu.SemaphoreType.DMA,
        ],
    )(src, dst)


def _edge_body(hp_hbm, lists_hbm, deg_hbm, zeros_hbm, acc_hbm,
               lbuf, degv, srcbuf, locbuf, rows, acc, semG):
    c = lax.axis_index("c")
    s = lax.axis_index("s")
    w = s * NC + c
    base = w * TPW

    iota16 = lax.iota(jnp.int32, 16)

    pltpu.sync_copy(zeros_hbm, acc)

    pltpu.sync_copy(deg_hbm.at[pl.ds(pl.multiple_of(base, 8), TPW)], degv)
    sv = jnp.zeros((16,), jnp.float32)
    for j in range(TPW // 16):
        sv = sv + degv[pl.ds(j * 16, 16)]
    m = jnp.sum(sv, axis=0).astype(jnp.int32)
    nb = jnp.right_shift(m + (GB - 1), 5)
    nrounds = jnp.right_shift(nb + (RV // GB - 1), 8)

    def unpack(k, par):
        boff = k * GB
        poff = par * GB
        for g in range(GB // 16):
            packed = lbuf[pl.ds(boff + g * 16, 16)]
            srcbuf[pl.ds(poff + g * 16, 16)] = jnp.right_shift(packed, 9)
            locbuf[pl.ds(poff + g * 16, 16)] = packed & 511

    def gdesc(par):
        idxs = srcbuf.at[pl.ds(par * GB, GB)]
        return pltpu.make_async_copy(hp_hbm.at[idxs], rows.at[par],
                                     semG.at[par])

    @pl.loop(0, nrounds)
    def _round(r):
        roff = pl.multiple_of(r * RV, 8)
        pltpu.sync_copy(lists_hbm.at[pl.ds(pl.multiple_of(w * LCAP + roff, 8), RV)], lbuf)
        nbr = jnp.minimum(nb - r * (RV // GB), RV // GB)

        for q in range(DEPTH):
            @pl.when(q < nbr)
            def _():
                unpack(q, q)
                gdesc(q).start()

        @pl.loop(0, nbr)
        def _batch(b):
            par = b & (DEPTH - 1)

            gdesc(par).wait()
            parv = jnp.full((16,), par, jnp.int32)
            for g in range(GB // 16):
                locv = locbuf[pl.ds(par * GB + g * 16, 16)]
                rowv = iota16 + g * 16

                @pl.loop(0, D // 16, unroll=2)
                def _cb(cb):
                    for colr in range(16):
                        cv = jnp.full((16,), cb * 16 + colr, jnp.int32)
                        vals = plsc.load_gather(rows, [parv, rowv, cv])
                        plsc.addupdate_scatter(acc, [locv, cv], vals)

            @pl.when(b + DEPTH < nbr)
            def _():
                unpack(b + DEPTH, par)
                gdesc(par).start()

    @pl.when(w < NW - 1)
    def _full():
        pltpu.sync_copy(acc.at[pl.ds(0, TPW)],
                        acc_hbm.at[pl.ds(base, TPW)])

    last = N - (NW - 1) * TPW

    @pl.when(w == NW - 1)
    def _last():
        pltpu.sync_copy(acc.at[pl.ds(0, last)],
                        acc_hbm.at[pl.ds(base, last)])


def _edge_call(hp, lists, deg, zeros_acc):
    return pl.kernel(
        _edge_body,
        out_type=jax.ShapeDtypeStruct((N, D), jnp.float32),
        mesh=_mesh(),
        compiler_params=_SC_PARAMS,
        scratch_types=[
            pltpu.VMEM((RV,), jnp.int32),
            pltpu.VMEM((TPW,), jnp.float32),
            pltpu.VMEM((DEPTH * GB,), jnp.int32),
            pltpu.VMEM((DEPTH * GB,), jnp.int32),
            pltpu.VMEM((DEPTH, GB, D), jnp.float32),
            pltpu.VMEM((ACCR, D), jnp.float32),
            pltpu.SemaphoreType.DMA((DEPTH,)),
        ],
    )(hp, lists, deg, zeros_acc)


def _dis_body(deg_ref, o_ref):
    d = deg_ref[0, :] + 1.0
    o_ref[0, :] = lax.rsqrt(d)


def _dis_call(deg):
    return pl.pallas_call(
        _dis_body,
        out_shape=jax.ShapeDtypeStruct((1, NPAD), jnp.float32),
    )(deg)


def _mm_in_body(x_ref, w_ref, b_ref, dis_ref, o_ref):
    h = jnp.dot(x_ref[...], w_ref[...], preferred_element_type=jnp.float32)
    h = jnp.maximum(h + b_ref[...], 0.0)
    o_ref[...] = h * dis_ref[...]


def _mm_in(x, W, b2, dis_col):
    return pl.pallas_call(
        _mm_in_body,
        grid=(GRID,),
        in_specs=[pl.BlockSpec((BR, D), lambda i: (i, 0)),
                  pl.BlockSpec((D, D), lambda i: (0, 0)),
                  pl.BlockSpec((1, D), lambda i: (0, 0)),
                  pl.BlockSpec((BR, 1), lambda i: (i, 0))],
        out_specs=pl.BlockSpec((BR, D), lambda i: (i, 0)),
        out_shape=jax.ShapeDtypeStruct((N, D), jnp.float32),
    )(x, W, b2, dis_col)


def _layer_body(acc_ref, hp_ref, dis_ref, w_ref, b_ref, o_ref):
    dis = dis_ref[...]
    g = (acc_ref[...] + hp_ref[...]) * dis
    h = jnp.dot(g, w_ref[...], preferred_element_type=jnp.float32)
    h = jnp.maximum(h + b_ref[...], 0.0)
    o_ref[...] = h * dis


def _layer(acc, hp, dis_col, W, b2):
    return pl.pallas_call(
        _layer_body,
        grid=(GRID,),
        in_specs=[pl.BlockSpec((BR, D), lambda i: (i, 0)),
                  pl.BlockSpec((BR, D), lambda i: (i, 0)),
                  pl.BlockSpec((BR, 1), lambda i: (i, 0)),
                  pl.BlockSpec((D, D), lambda i: (0, 0)),
                  pl.BlockSpec((1, D), lambda i: (0, 0))],
        out_specs=pl.BlockSpec((BR, D), lambda i: (i, 0)),
        out_shape=jax.ShapeDtypeStruct((N, D), jnp.float32),
    )(acc, hp, dis_col, W, b2)


def _final_body(acc_ref, hp_ref, dis_ref, w_ref, b_ref, wh_ref, bh_ref, o_ref):
    dis = dis_ref[...]
    g = (acc_ref[...] + hp_ref[...]) * dis
    h = jnp.dot(g, w_ref[...], preferred_element_type=jnp.float32)
    h = jnp.maximum(h + b_ref[...], 0.0)
    t = jnp.dot(h, wh_ref[...], preferred_element_type=jnp.float32) + bh_ref[...]
    lane = lax.broadcasted_iota(jnp.int32, t.shape, 1)
    is_c = lane < 3
    m = jnp.max(jnp.where(is_c, t, -1e30), axis=1, keepdims=True)
    e = jnp.where(is_c, jnp.exp(t - m), 0.0)
    cls = e / jnp.sum(e, axis=1, keepdims=True)
    score = 1.0 / (1.0 + jnp.exp(-t))
    o_ref[...] = jnp.where(is_c, cls, jnp.where(lane == 3, score, 0.0))


def _final(acc, hp, dis_col, W, b2, Wh, bh):
    return pl.pallas_call(
        _final_body,
        grid=(GRID,),
        in_specs=[pl.BlockSpec((BR, D), lambda i: (i, 0)),
                  pl.BlockSpec((BR, D), lambda i: (i, 0)),
                  pl.BlockSpec((BR, 1), lambda i: (i, 0)),
                  pl.BlockSpec((D, D), lambda i: (0, 0)),
                  pl.BlockSpec((1, D), lambda i: (0, 0)),
                  pl.BlockSpec((D, 128), lambda i: (0, 0)),
                  pl.BlockSpec((1, 128), lambda i: (0, 0))],
        out_specs=pl.BlockSpec((BR, 128), lambda i: (i, 0)),
        out_shape=jax.ShapeDtypeStruct((N, 128), jnp.float32),
    )(acc, hp, dis_col, W, b2, Wh, bh)


def kernel(x, edge_index, W_in, b_in, W1, b1, W2, b2, Wc, bc, Ws, bs):
    src = edge_index[0]
    dst = edge_index[1]

    lists, deg = _route_call(src, dst)
    dis_row = _dis_call(deg.reshape(1, NPAD))
    dis_col = dis_row.reshape(NPAD, 1)[:N]

    zeros_acc = jnp.zeros((ACCR, D), jnp.float32)
    h0p = _mm_in(x, W_in, b_in.reshape(1, D), dis_col)
    acc1 = _edge_call(h0p, lists, deg, zeros_acc)
    h1p = _layer(acc1, h0p, dis_col, W1, b1.reshape(1, D))
    acc2 = _edge_call(h1p, lists, deg, zeros_acc)

    Wh = jnp.zeros((D, 128), jnp.float32).at[:, :3].set(Wc).at[:, 3:4].set(Ws)
    bh = jnp.zeros((1, 128), jnp.float32).at[0, :3].set(bc).at[0, 3].set(bs[0])
    out128 = _final(acc2, h1p, dis_col, W2, b2.reshape(1, D), Wh, bh)
    return out128[:, :3], out128[:, 3:4]


# speedup vs baseline: 3.9839x; 3.9839x over previous
import jax
import jax.numpy as jnp
from jax import lax
from jax.experimental import pallas as pl
from jax.experimental.pallas import tpu as pltpu
from jax.experimental.pallas import tpu_sc as plsc

N = 10000
E = 160000
D = 256
NPAD = 10240
NC = 2
NS = 16
NW = NC * NS
TPW = NPAD // NW
ACCR = TPW + 8
CH = 1600
NCHE = E // CH
NV = CH // 16
CL = 6480
FB = 2048
SB = 64
RV = 8192
GB = 32
DEPTH = 4
LCAP = E + RV

BR = 400
GRID = N // BR


def _mesh():
    return plsc.VectorSubcoreMesh(core_axis_name="c", subcore_axis_name="s")


_SC_PARAMS = pltpu.CompilerParams(needs_layout_passes=False)


def _route_body(src_hbm, dst_hbm, lists_hbm, deg_hbm,
                sb0, db0, sb1, db1, pklist, hist, semA, semB):
    c = lax.axis_index("c")
    s = lax.axis_index("s")
    w = s * NC + c
    base = w * TPW
    lbase = w * LCAP

    iota16 = lax.iota(jnp.int32, 16)
    trash16 = TPW + (iota16 & 7)
    zeros16 = jnp.zeros((16,), jnp.float32)
    ones16 = jnp.ones((16,), jnp.float32)

    for j in range(21):
        hist[pl.ds(j * 16, 16)] = zeros16

    def stage_start(ch, sbuf, dbuf, sem):
        eoff = pl.multiple_of(ch * CH, 8)
        pltpu.async_copy(src_hbm.at[pl.ds(eoff, CH)], sbuf, sem)
        pltpu.async_copy(dst_hbm.at[pl.ds(eoff, CH)], dbuf, sem)

    def stage_wait(sbuf, dbuf, sem):
        pltpu.make_async_copy(src_hbm.at[pl.ds(0, CH)], sbuf, sem).wait()
        pltpu.make_async_copy(dst_hbm.at[pl.ds(0, CH)], dbuf, sem).wait()

    def scan_chunk(sbuf, dbuf, cnt0):
        def scan(i, cnt):
            d = dbuf[pl.ds(i * 16, 16)]
            sv = sbuf[pl.ds(i * 16, 16)]
            loc = d - base
            mask = (loc >= 0) & (loc < TPW)
            packed = jnp.bitwise_or(jnp.left_shift(sv, 9), loc & 511)
            plsc.store_compressed(pklist.at[pl.ds(cnt, 16)], packed, mask=mask)
            locm = jnp.where(mask, loc, TPW)
            plsc.addupdate_scatter(hist, [locm], ones16, mask=mask)
            pc = plsc.all_reduce_population_count(mask)
            return cnt + pc[0]

        return pl.loop(0, NV, init_carry=cnt0, unroll=4)(scan)

    def maybe_flush(m, tot):
        cond = m > CL - 2 * CH
        nfb = jnp.where(cond, jnp.right_shift(m, 11), 0)

        @pl.when(cond)
        def _():
            @pl.loop(0, nfb)
            def _f(k):
                off = pl.multiple_of(k * FB, 8)
                pltpu.sync_copy(pklist.at[pl.ds(off, FB)],
                                lists_hbm.at[pl.ds(pl.multiple_of(lbase + tot + off, 8), FB)])

            rem0 = nfb * FB
            nmv = jnp.right_shift((m - rem0) + 15, 4)

            @pl.loop(0, nmv)
            def _mv(j):
                pklist[pl.ds(pl.multiple_of(j * 16, 8), 16)] = (
                    pklist[pl.ds(pl.multiple_of(rem0 + j * 16, 8), 16)])

        return jnp.where(cond, m - nfb * FB, m), tot + nfb * FB

    stage_start(0, sb0, db0, semA)

    @pl.loop(0, NCHE // 2, init_carry=(jnp.int32(0), jnp.int32(0)))
    def _pair(p, carry):
        m, tot = carry
        ch0 = p * 2
        stage_wait(sb0, db0, semA)
        stage_start(ch0 + 1, sb1, db1, semB)
        m = scan_chunk(sb0, db0, m)
        stage_wait(sb1, db1, semB)

        @pl.when(ch0 + 2 < NCHE)
        def _():
            stage_start(ch0 + 2, sb0, db0, semA)

        m = scan_chunk(sb1, db1, m)
        return maybe_flush(m, tot)

    m, tot = _pair

    for kpad in range(SB // 16):
        pklist[pl.ds(m + kpad * 16, 16)] = trash16
    n64 = jnp.right_shift(m + (SB - 1), 6)

    @pl.loop(0, n64)
    def _ff(k):
        off = pl.multiple_of(k * SB, 8)
        pltpu.sync_copy(pklist.at[pl.ds(off, SB)],
                        lists_hbm.at[pl.ds(pl.multiple_of(lbase + tot + off, 8), SB)])

    pltpu.sync_copy(hist.at[pl.ds(0, TPW)],
                    deg_hbm.at[pl.ds(pl.multiple_of(base, 8), TPW)])


def _route_call(src, dst):
    return pl.kernel(
        _route_body,
        out_type=(jax.ShapeDtypeStruct((NW * LCAP,), jnp.int32),
                  jax.ShapeDtypeStruct((NPAD,), jnp.float32)),
        mesh=_mesh(),
        compiler_params=_SC_PARAMS,
        scratch_types=[
            pltpu.VMEM((CH,), jnp.int32),
            pltpu.VMEM((CH,), jnp.int32),
            pltpu.VMEM((CH,), jnp.int32),
            pltpu.VMEM((CH,), jnp.int32),
            pltpu.VMEM((CL,), jnp.int32),
            pltpu.VMEM((336,), jnp.float32),
            pltpu.SemaphoreType.DMA,
            pltpu.SemaphoreType.DMA,
        ],
    )(src, dst)


def _edge_body(hp_hbm, lists_hbm, deg_hbm, zeros_hbm, acc_hbm,
               lbuf, degv, srcbuf, locbuf, rows, acc, semG):
    c = lax.axis_index("c")
    s = lax.axis_index("s")
    w = s * NC + c
    base = w * TPW

    iota16 = lax.iota(jnp.int32, 16)

    pltpu.sync_copy(zeros_hbm, acc)

    pltpu.sync_copy(deg_hbm.at[pl.ds(pl.multiple_of(base, 8), TPW)], degv)
    sv = jnp.zeros((16,), jnp.float32)
    for j in range(TPW // 16):
        sv = sv + degv[pl.ds(j * 16, 16)]
    m = jnp.sum(sv, axis=0).astype(jnp.int32)
    nb = jnp.right_shift(m + (GB - 1), 5)
    nrounds = jnp.right_shift(nb + (RV // GB - 1), 8)

    def unpack(k, par):
        boff = k * GB
        poff = par * GB
        for g in range(GB // 16):
            packed = lbuf[pl.ds(boff + g * 16, 16)]
            srcbuf[pl.ds(poff + g * 16, 16)] = jnp.right_shift(packed, 9)
            locbuf[pl.ds(poff + g * 16, 16)] = packed & 511

    def gdesc(par):
        idxs = srcbuf.at[pl.ds(par * GB, GB)]
        return pltpu.make_async_copy(hp_hbm.at[idxs], rows.at[par],
                                     semG.at[par])

    @pl.loop(0, nrounds)
    def _round(r):
        roff = pl.multiple_of(r * RV, 8)
        pltpu.sync_copy(lists_hbm.at[pl.ds(pl.multiple_of(w * LCAP + roff, 8), RV)], lbuf)
        nbr = jnp.minimum(nb - r * (RV // GB), RV // GB)

        for q in range(DEPTH):
            @pl.when(q < nbr)
            def _():
                unpack(q, q)
                gdesc(q).start()

        @pl.loop(0, nbr)
        def _batch(b):
            par = b & (DEPTH - 1)

            gdesc(par).wait()
            parv = jnp.full((16,), par, jnp.int32)
            for g in range(GB // 16):
                locv = locbuf[pl.ds(par * GB + g * 16, 16)]
                rowv = iota16 + g * 16

                @pl.loop(0, D // 16, unroll=2)
                def _cb(cb):
                    for colr in range(16):
                        cv = (iota16 + (cb * 16 + colr)) & (D - 1)
                        vals = plsc.load_gather(rows, [parv, rowv, cv])
                        plsc.addupdate_scatter(acc, [locv, cv], vals)

            @pl.when(b + DEPTH < nbr)
            def _():
                unpack(b + DEPTH, par)
                gdesc(par).start()

    @pl.when(w < NW - 1)
    def _full():
        pltpu.sync_copy(acc.at[pl.ds(0, TPW)],
                        acc_hbm.at[pl.ds(base, TPW)])

    last = N - (NW - 1) * TPW

    @pl.when(w == NW - 1)
    def _last():
        pltpu.sync_copy(acc.at[pl.ds(0, last)],
                        acc_hbm.at[pl.ds(base, last)])


def _edge_call(hp, lists, deg, zeros_acc):
    return pl.kernel(
        _edge_body,
        out_type=jax.ShapeDtypeStruct((N, D), jnp.float32),
        mesh=_mesh(),
        compiler_params=_SC_PARAMS,
        scratch_types=[
            pltpu.VMEM((RV,), jnp.int32),
            pltpu.VMEM((TPW,), jnp.float32),
            pltpu.VMEM((DEPTH * GB,), jnp.int32),
            pltpu.VMEM((DEPTH * GB,), jnp.int32),
            pltpu.VMEM((DEPTH, GB, D), jnp.float32),
            pltpu.VMEM((ACCR, D), jnp.float32),
            pltpu.SemaphoreType.DMA((DEPTH,)),
        ],
    )(hp, lists, deg, zeros_acc)


def _dis_body(deg_ref, o_ref):
    d = deg_ref[0, :] + 1.0
    o_ref[0, :] = lax.rsqrt(d)


def _dis_call(deg):
    return pl.pallas_call(
        _dis_body,
        out_shape=jax.ShapeDtypeStruct((1, NPAD), jnp.float32),
    )(deg)


def _mm_in_body(x_ref, w_ref, b_ref, dis_ref, o_ref):
    h = jnp.dot(x_ref[...], w_ref[...], preferred_element_type=jnp.float32)
    h = jnp.maximum(h + b_ref[...], 0.0)
    o_ref[...] = h * dis_ref[...]


def _mm_in(x, W, b2, dis_col):
    return pl.pallas_call(
        _mm_in_body,
        grid=(GRID,),
        in_specs=[pl.BlockSpec((BR, D), lambda i: (i, 0)),
                  pl.BlockSpec((D, D), lambda i: (0, 0)),
                  pl.BlockSpec((1, D), lambda i: (0, 0)),
                  pl.BlockSpec((BR, 1), lambda i: (i, 0))],
        out_specs=pl.BlockSpec((BR, D), lambda i: (i, 0)),
        out_shape=jax.ShapeDtypeStruct((N, D), jnp.float32),
    )(x, W, b2, dis_col)


def _layer_body(acc_ref, hp_ref, dis_ref, w_ref, b_ref, o_ref):
    dis = dis_ref[...]
    g = (acc_ref[...] + hp_ref[...]) * dis
    h = jnp.dot(g, w_ref[...], preferred_element_type=jnp.float32)
    h = jnp.maximum(h + b_ref[...], 0.0)
    o_ref[...] = h * dis


def _layer(acc, hp, dis_col, W, b2):
    return pl.pallas_call(
        _layer_body,
        grid=(GRID,),
        in_specs=[pl.BlockSpec((BR, D), lambda i: (i, 0)),
                  pl.BlockSpec((BR, D), lambda i: (i, 0)),
                  pl.BlockSpec((BR, 1), lambda i: (i, 0)),
                  pl.BlockSpec((D, D), lambda i: (0, 0)),
                  pl.BlockSpec((1, D), lambda i: (0, 0))],
        out_specs=pl.BlockSpec((BR, D), lambda i: (i, 0)),
        out_shape=jax.ShapeDtypeStruct((N, D), jnp.float32),
    )(acc, hp, dis_col, W, b2)


def _final_body(acc_ref, hp_ref, dis_ref, w_ref, b_ref, wh_ref, bh_ref, o_ref):
    dis = dis_ref[...]
    g = (acc_ref[...] + hp_ref[...]) * dis
    h = jnp.dot(g, w_ref[...], preferred_element_type=jnp.float32)
    h = jnp.maximum(h + b_ref[...], 0.0)
    t = jnp.dot(h, wh_ref[...], preferred_element_type=jnp.float32) + bh_ref[...]
    lane = lax.broadcasted_iota(jnp.int32, t.shape, 1)
    is_c = lane < 3
    m = jnp.max(jnp.where(is_c, t, -1e30), axis=1, keepdims=True)
    e = jnp.where(is_c, jnp.exp(t - m), 0.0)
    cls = e / jnp.sum(e, axis=1, keepdims=True)
    score = 1.0 / (1.0 + jnp.exp(-t))
    o_ref[...] = jnp.where(is_c, cls, jnp.where(lane == 3, score, 0.0))


def _final(acc, hp, dis_col, W, b2, Wh, bh):
    return pl.pallas_call(
        _final_body,
        grid=(GRID,),
        in_specs=[pl.BlockSpec((BR, D), lambda i: (i, 0)),
                  pl.BlockSpec((BR, D), lambda i: (i, 0)),
                  pl.BlockSpec((BR, 1), lambda i: (i, 0)),
                  pl.BlockSpec((D, D), lambda i: (0, 0)),
                  pl.BlockSpec((1, D), lambda i: (0, 0)),
                  pl.BlockSpec((D, 128), lambda i: (0, 0)),
                  pl.BlockSpec((1, 128), lambda i: (0, 0))],
        out_specs=pl.BlockSpec((BR, 128), lambda i: (i, 0)),
        out_shape=jax.ShapeDtypeStruct((N, 128), jnp.float32),
    )(acc, hp, dis_col, W, b2, Wh, bh)


def kernel(x, edge_index, W_in, b_in, W1, b1, W2, b2, Wc, bc, Ws, bs):
    src = edge_index[0]
    dst = edge_index[1]

    lists, deg = _route_call(src, dst)
    dis_row = _dis_call(deg.reshape(1, NPAD))
    dis_col = dis_row.reshape(NPAD, 1)[:N]

    zeros_acc = jnp.zeros((ACCR, D), jnp.float32)
    h0p = _mm_in(x, W_in, b_in.reshape(1, D), dis_col)
    acc1 = _edge_call(h0p, lists, deg, zeros_acc)
    h1p = _layer(acc1, h0p, dis_col, W1, b1.reshape(1, D))
    acc2 = _edge_call(h1p, lists, deg, zeros_acc)

    Wh = jnp.zeros((D, 128), jnp.float32).at[:, :3].set(Wc).at[:, 3:4].set(Ws)
    bh = jnp.zeros((1, 128), jnp.float32).at[0, :3].set(bc).at[0, 3].set(bs[0])
    out128 = _final(acc2, h1p, dis_col, W2, b2.reshape(1, D), Wh, bh)
    return out128[:, :3], out128[:, 3:4]
